# Initial kernel scaffold; baseline (speedup 1.0000x reference)
import jax
import jax.numpy as jnp
from jax.experimental import pallas as pl


def kernel(pred_eps_x, target_eps_x, weights_x, pred_eps_h, eps_h, pred_symmetric_vector, symmetric_matrix_vector, batch_idx):
    raise NotImplementedError("write your pallas kernel here")


# speedup vs baseline: 2.2347x; 2.2347x over previous
import functools

import jax
import jax.numpy as jnp
from jax import lax
from jax.experimental import pallas as pl
from jax.experimental.pallas import tpu as pltpu
from jax.experimental.pallas import tpu_sc as plsc

_B = 2048
_N = 262144
_H = 100

_ROWS = 2048


def _rowsum_body(ph, eh, px, tx, w, out):
    dh = eh[...] - ph[...]
    sh = jnp.sum(dh * dh, axis=1, keepdims=True)
    dx = tx[...] - px[...]
    sx = jnp.sum(dx * dx, axis=1, keepdims=True) * w[...]
    out[...] = sh + sx


def _per_atom_sums(ph, eh, px, tx, w):
    return pl.pallas_call(
        _rowsum_body,
        grid=(_N // _ROWS,),
        in_specs=[
            pl.BlockSpec((_ROWS, _H), lambda i: (i, 0)),
            pl.BlockSpec((_ROWS, _H), lambda i: (i, 0)),
            pl.BlockSpec((_ROWS, 3), lambda i: (i, 0)),
            pl.BlockSpec((_ROWS, 3), lambda i: (i, 0)),
            pl.BlockSpec((_ROWS, 1), lambda i: (i, 0)),
        ],
        out_specs=pl.BlockSpec((_ROWS, 1), lambda i: (i, 0)),
        out_shape=jax.ShapeDtypeStruct((_N, 1), jnp.float32),
        compiler_params=pltpu.CompilerParams(
            dimension_semantics=("arbitrary",)),
    )(ph, eh, px, tx, w)


_NC, _NS, _L = 2, 16, 16
_NW = _NC * _NS
_LANES = 128
_ROWS_TOTAL = _N // _LANES
_ROWS_W = _ROWS_TOTAL // _NW


def _seg_body(s_hbm, idx_hbm, acc_out, cnt_out,
              s_v, idx_v, ones_v, zero_v, acc_sh, cnt_sh):
    cid = lax.axis_index("c")
    sid = lax.axis_index("s")
    wid = sid * _NC + cid

    pltpu.sync_copy(s_hbm.at[pl.ds(wid * _ROWS_W, _ROWS_W)], s_v)
    pltpu.sync_copy(idx_hbm.at[pl.ds(wid * _ROWS_W, _ROWS_W)], idx_v)

    def _fill_ones(k, _):
        ones_v[pl.ds(k * _L, _L)] = jnp.full((_L,), 1.0, jnp.float32)
        return 0
    lax.fori_loop(0, _LANES // _L, _fill_ones, 0)

    @pl.when(sid == 0)
    def _init():
        def _fill_zero(k, _):
            zero_v[pl.ds(k * _L, _L)] = jnp.zeros((_L,), jnp.float32)
            return 0
        lax.fori_loop(0, _B // _L, _fill_zero, 0)
        pltpu.sync_copy(zero_v, acc_sh)
        pltpu.sync_copy(zero_v, cnt_sh)

    plsc.subcore_barrier()

    def _scatter(j, _):
        pltpu.sync_copy(s_v.at[j], acc_sh.at[idx_v.at[j]], add=True)
        pltpu.sync_copy(ones_v, cnt_sh.at[idx_v.at[j]], add=True)
        return 0
    lax.fori_loop(0, _ROWS_W, _scatter, 0)

    plsc.subcore_barrier()

    @pl.when(sid == 0)
    def _publish():
        pltpu.sync_copy(acc_sh, acc_out.at[cid])
        pltpu.sync_copy(cnt_sh, cnt_out.at[cid])


def _segment_partials(s2d, idx2d):
    mesh = plsc.VectorSubcoreMesh(
        core_axis_name="c", subcore_axis_name="s",
        num_cores=_NC, num_subcores=_NS)
    fn = functools.partial(
        pl.kernel,
        out_type=[jax.ShapeDtypeStruct((_NC, _B), jnp.float32),
                  jax.ShapeDtypeStruct((_NC, _B), jnp.float32)],
        mesh=mesh,
        scratch_types=[
            pltpu.VMEM((_ROWS_W, _LANES), jnp.float32),
            pltpu.VMEM((_ROWS_W, _LANES), jnp.int32),
            pltpu.VMEM((_LANES,), jnp.float32),
            pltpu.VMEM((_B,), jnp.float32),
            pltpu.VMEM_SHARED((_B,), jnp.float32),
            pltpu.VMEM_SHARED((_B,), jnp.float32),
        ],
    )(_seg_body)
    return fn(s2d, idx2d)


def _combine_body(acc, cnt, psv, smv, out):
    seg = acc[0:1, :] + acc[1:2, :]
    c = jnp.maximum(cnt[0:1, :] + cnt[1:2, :], 1.0)
    part = jnp.sum(seg / c)
    d = psv[...] - smv[...]
    errl = jnp.sum(d * d) / (smv.shape[0] * smv.shape[1])
    out[0, 0] = part / _B + errl


def _combine(acc, cnt, psv, smv):
    return pl.pallas_call(
        _combine_body,
        in_specs=[pl.BlockSpec(memory_space=pltpu.VMEM)] * 4,
        out_specs=pl.BlockSpec(memory_space=pltpu.SMEM),
        out_shape=jax.ShapeDtypeStruct((1, 1), jnp.float32),
    )(acc, cnt, psv, smv)


def kernel(pred_eps_x, target_eps_x, weights_x, pred_eps_h, eps_h,
           pred_symmetric_vector, symmetric_matrix_vector, batch_idx):
    s = _per_atom_sums(pred_eps_h, eps_h, pred_eps_x, target_eps_x, weights_x)
    s2d = s.reshape(_ROWS_TOTAL, _LANES)
    idx2d = batch_idx.reshape(_ROWS_TOTAL, _LANES)
    acc, cnt = _segment_partials(s2d, idx2d)
    out = _combine(acc, cnt, pred_symmetric_vector, symmetric_matrix_vector)
    return out[0, 0]


# speedup vs baseline: 2.5902x; 1.1591x over previous
import functools

import jax
import jax.numpy as jnp
from jax import lax
from jax.experimental import pallas as pl
from jax.experimental.pallas import tpu as pltpu
from jax.experimental.pallas import tpu_sc as plsc

_B = 2048
_N = 262144
_H = 100

_ROWS = 8192


def _rowsum_body(ph, eh, px, tx, w, out):
    dh = eh[...] - ph[...]
    sh = jnp.sum(dh * dh, axis=1)
    dx = tx[...] - px[...]
    sx = jnp.sum(dx * dx, axis=1) * w[:, 0]
    s = sh + sx
    out[...] = s.reshape(_ROWS // 128, 128)


def _per_atom_sums(ph, eh, px, tx, w):
    return pl.pallas_call(
        _rowsum_body,
        grid=(_N // _ROWS,),
        in_specs=[
            pl.BlockSpec((_ROWS, _H), lambda i: (i, 0)),
            pl.BlockSpec((_ROWS, _H), lambda i: (i, 0)),
            pl.BlockSpec((_ROWS, 3), lambda i: (i, 0)),
            pl.BlockSpec((_ROWS, 3), lambda i: (i, 0)),
            pl.BlockSpec((_ROWS, 1), lambda i: (i, 0)),
        ],
        out_specs=pl.BlockSpec((_ROWS // 128, 128), lambda i: (i, 0)),
        out_shape=jax.ShapeDtypeStruct((_N // 128, 128), jnp.float32),
        compiler_params=pltpu.CompilerParams(
            dimension_semantics=("arbitrary",)),
    )(ph, eh, px, tx, w)


_NC, _NS, _L = 2, 16, 16
_NW = _NC * _NS
_LANES = 128
_ROWS_TOTAL = _N // _LANES
_ROWS_W = _ROWS_TOTAL // _NW


def _seg_body(s_hbm, idx_hbm, acc_out, cnt_out,
              s_v, idx_v, ones_v, zero_v, acc_sh, cnt_sh):
    cid = lax.axis_index("c")
    sid = lax.axis_index("s")
    wid = sid * _NC + cid

    pltpu.sync_copy(s_hbm.at[pl.ds(wid * _ROWS_W, _ROWS_W)], s_v)
    pltpu.sync_copy(idx_hbm.at[pl.ds(wid * _ROWS_W, _ROWS_W)], idx_v)

    def _fill_ones(k, _):
        ones_v[pl.ds(k * _L, _L)] = jnp.full((_L,), 1.0, jnp.float32)
        return 0
    lax.fori_loop(0, _LANES // _L, _fill_ones, 0)

    @pl.when(sid == 0)
    def _init():
        def _fill_zero(k, _):
            zero_v[pl.ds(k * _L, _L)] = jnp.zeros((_L,), jnp.float32)
            return 0
        lax.fori_loop(0, _B // _L, _fill_zero, 0)
        pltpu.sync_copy(zero_v, acc_sh)
        pltpu.sync_copy(zero_v, cnt_sh)

    plsc.subcore_barrier()

    def _scatter(j, _):
        pltpu.sync_copy(s_v.at[j], acc_sh.at[idx_v.at[j]], add=True)
        pltpu.sync_copy(ones_v, cnt_sh.at[idx_v.at[j]], add=True)
        return 0
    lax.fori_loop(0, _ROWS_W, _scatter, 0)

    plsc.subcore_barrier()

    @pl.when(sid == 0)
    def _publish():
        pltpu.sync_copy(acc_sh, acc_out.at[cid])
        pltpu.sync_copy(cnt_sh, cnt_out.at[cid])


def _segment_partials(s2d, idx2d):
    mesh = plsc.VectorSubcoreMesh(
        core_axis_name="c", subcore_axis_name="s",
        num_cores=_NC, num_subcores=_NS)
    fn = functools.partial(
        pl.kernel,
        out_type=[jax.ShapeDtypeStruct((_NC, _B), jnp.float32),
                  jax.ShapeDtypeStruct((_NC, _B), jnp.float32)],
        mesh=mesh,
        scratch_types=[
            pltpu.VMEM((_ROWS_W, _LANES), jnp.float32),
            pltpu.VMEM((_ROWS_W, _LANES), jnp.int32),
            pltpu.VMEM((_LANES,), jnp.float32),
            pltpu.VMEM((_B,), jnp.float32),
            pltpu.VMEM_SHARED((_B,), jnp.float32),
            pltpu.VMEM_SHARED((_B,), jnp.float32),
        ],
    )(_seg_body)
    return fn(s2d, idx2d)


def _combine_body(acc, cnt, psv, smv, out):
    seg = acc[0:1, :] + acc[1:2, :]
    c = jnp.maximum(cnt[0:1, :] + cnt[1:2, :], 1.0)
    part = jnp.sum(seg / c)
    d = psv[...] - smv[...]
    errl = jnp.sum(d * d) / (smv.shape[0] * smv.shape[1])
    out[0, 0] = part / _B + errl


def _combine(acc, cnt, psv, smv):
    return pl.pallas_call(
        _combine_body,
        in_specs=[pl.BlockSpec(memory_space=pltpu.VMEM)] * 4,
        out_specs=pl.BlockSpec(memory_space=pltpu.SMEM),
        out_shape=jax.ShapeDtypeStruct((1, 1), jnp.float32),
    )(acc, cnt, psv, smv)


def kernel(pred_eps_x, target_eps_x, weights_x, pred_eps_h, eps_h,
           pred_symmetric_vector, symmetric_matrix_vector, batch_idx):
    s2d = _per_atom_sums(pred_eps_h, eps_h, pred_eps_x, target_eps_x,
                         weights_x)
    idx2d = batch_idx.reshape(_ROWS_TOTAL, _LANES)
    acc, cnt = _segment_partials(s2d, idx2d)
    out = _combine(acc, cnt, pred_symmetric_vector, symmetric_matrix_vector)
    return out[0, 0]


# speedup vs baseline: 12.0955x; 4.6696x over previous
import functools

import jax
import jax.numpy as jnp
from jax import lax
from jax.experimental import pallas as pl
from jax.experimental.pallas import tpu as pltpu
from jax.experimental.pallas import tpu_sc as plsc

_B = 2048
_N = 262144
_H = 100

_ROWS = 8192


def _rowsum_body(phT, ehT, pxT, txT, wT, out):
    dh = ehT[...] - phT[...]
    sh = jnp.sum(dh * dh, axis=0)
    dx = txT[...] - pxT[...]
    sx = jnp.sum(dx * dx, axis=0) * wT[0, :]
    out[...] = (sh + sx).reshape(_ROWS // 128, 128)


def _per_atom_sums(ph, eh, px, tx, w):
    return pl.pallas_call(
        _rowsum_body,
        grid=(_N // _ROWS,),
        in_specs=[
            pl.BlockSpec((_H, _ROWS), lambda i: (0, i)),
            pl.BlockSpec((_H, _ROWS), lambda i: (0, i)),
            pl.BlockSpec((3, _ROWS), lambda i: (0, i)),
            pl.BlockSpec((3, _ROWS), lambda i: (0, i)),
            pl.BlockSpec((1, _ROWS), lambda i: (0, i)),
        ],
        out_specs=pl.BlockSpec((_ROWS // 128, 128), lambda i: (i, 0)),
        out_shape=jax.ShapeDtypeStruct((_N // 128, 128), jnp.float32),
        compiler_params=pltpu.CompilerParams(
            dimension_semantics=("arbitrary",)),
    )(ph.T, eh.T, px.T, tx.T, w.T)


_NC, _NS, _L = 2, 16, 16
_NW = _NC * _NS
_LANES = 128
_ROWS_TOTAL = _N // _LANES
_ROWS_W = _ROWS_TOTAL // _NW


def _seg_body(s_hbm, idx_hbm, acc_out, cnt_out,
              s_v, idx_v, ones_v, zero_v, acc_sh, cnt_sh):
    cid = lax.axis_index("c")
    sid = lax.axis_index("s")
    wid = sid * _NC + cid

    pltpu.sync_copy(s_hbm.at[pl.ds(wid * _ROWS_W, _ROWS_W)], s_v)
    pltpu.sync_copy(idx_hbm.at[pl.ds(wid * _ROWS_W, _ROWS_W)], idx_v)

    def _fill_ones(k, _):
        ones_v[pl.ds(k * _L, _L)] = jnp.full((_L,), 1.0, jnp.float32)
        return 0
    lax.fori_loop(0, _LANES // _L, _fill_ones, 0)

    @pl.when(sid == 0)
    def _init():
        def _fill_zero(k, _):
            zero_v[pl.ds(k * _L, _L)] = jnp.zeros((_L,), jnp.float32)
            return 0
        lax.fori_loop(0, _B // _L, _fill_zero, 0)
        pltpu.sync_copy(zero_v, acc_sh)
        pltpu.sync_copy(zero_v, cnt_sh)

    plsc.subcore_barrier()

    def _scatter(j, _):
        pltpu.sync_copy(s_v.at[j], acc_sh.at[idx_v.at[j]], add=True)
        pltpu.sync_copy(ones_v, cnt_sh.at[idx_v.at[j]], add=True)
        return 0
    lax.fori_loop(0, _ROWS_W, _scatter, 0)

    plsc.subcore_barrier()

    @pl.when(sid == 0)
    def _publish():
        pltpu.sync_copy(acc_sh, acc_out.at[cid])
        pltpu.sync_copy(cnt_sh, cnt_out.at[cid])


def _segment_partials(s2d, idx2d):
    mesh = plsc.VectorSubcoreMesh(
        core_axis_name="c", subcore_axis_name="s",
        num_cores=_NC, num_subcores=_NS)
    fn = functools.partial(
        pl.kernel,
        out_type=[jax.ShapeDtypeStruct((_NC, _B), jnp.float32),
                  jax.ShapeDtypeStruct((_NC, _B), jnp.float32)],
        mesh=mesh,
        scratch_types=[
            pltpu.VMEM((_ROWS_W, _LANES), jnp.float32),
            pltpu.VMEM((_ROWS_W, _LANES), jnp.int32),
            pltpu.VMEM((_LANES,), jnp.float32),
            pltpu.VMEM((_B,), jnp.float32),
            pltpu.VMEM_SHARED((_B,), jnp.float32),
            pltpu.VMEM_SHARED((_B,), jnp.float32),
        ],
    )(_seg_body)
    return fn(s2d, idx2d)


def _combine_body(acc, cnt, psv, smv, out):
    seg = acc[0:1, :] + acc[1:2, :]
    c = jnp.maximum(cnt[0:1, :] + cnt[1:2, :], 1.0)
    part = jnp.sum(seg / c)
    d = psv[...] - smv[...]
    errl = jnp.sum(d * d) / (smv.shape[0] * smv.shape[1])
    out[0, 0] = part / _B + errl


def _combine(acc, cnt, psv, smv):
    return pl.pallas_call(
        _combine_body,
        in_specs=[pl.BlockSpec(memory_space=pltpu.VMEM)] * 4,
        out_specs=pl.BlockSpec(memory_space=pltpu.SMEM),
        out_shape=jax.ShapeDtypeStruct((1, 1), jnp.float32),
    )(acc, cnt, psv, smv)


def kernel(pred_eps_x, target_eps_x, weights_x, pred_eps_h, eps_h,
           pred_symmetric_vector, symmetric_matrix_vector, batch_idx):
    s2d = _per_atom_sums(pred_eps_h, eps_h, pred_eps_x, target_eps_x,
                         weights_x)
    idx2d = batch_idx.reshape(_ROWS_TOTAL, _LANES)
    acc, cnt = _segment_partials(s2d, idx2d)
    out = _combine(acc, cnt, pred_symmetric_vector, symmetric_matrix_vector)
    return out[0, 0]


# speedup vs baseline: 12.3121x; 1.0179x over previous
import functools

import jax
import jax.numpy as jnp
from jax import lax
from jax.experimental import pallas as pl
from jax.experimental.pallas import tpu as pltpu
from jax.experimental.pallas import tpu_sc as plsc

_B = 2048
_N = 262144
_H = 100

_ROWS = 8192


def _rowsum_body(phT, ehT, pxT, txT, wT, out):
    dh = ehT[...] - phT[...]
    sh = jnp.sum(dh * dh, axis=0)
    dx = txT[...] - pxT[...]
    sx = jnp.sum(dx * dx, axis=0) * wT[0, :]
    out[...] = (sh + sx).reshape(_ROWS // 128, 128)


def _per_atom_sums(ph, eh, px, tx, w):
    return pl.pallas_call(
        _rowsum_body,
        grid=(_N // _ROWS,),
        in_specs=[
            pl.BlockSpec((_H, _ROWS), lambda i: (0, i)),
            pl.BlockSpec((_H, _ROWS), lambda i: (0, i)),
            pl.BlockSpec((3, _ROWS), lambda i: (0, i)),
            pl.BlockSpec((3, _ROWS), lambda i: (0, i)),
            pl.BlockSpec((1, _ROWS), lambda i: (0, i)),
        ],
        out_specs=pl.BlockSpec((_ROWS // 128, 128), lambda i: (i, 0)),
        out_shape=jax.ShapeDtypeStruct((_N // 128, 128), jnp.float32),
        compiler_params=pltpu.CompilerParams(
            dimension_semantics=("arbitrary",)),
    )(ph.T, eh.T, px.T, tx.T, w.T)


_NC, _NS, _L = 2, 16, 16
_NW = _NC * _NS
_LANES = 128
_ROWS_TOTAL = _N // _LANES
_ROWS_W = _ROWS_TOTAL // _NW


_FIRE = 8


def _seg_body(s_hbm, idx_hbm, acc_out, cnt_out,
              s_v, idx_v, ones_v, zero_v, acc_sh, cnt_sh, sem_in, sem_sc):
    cid = lax.axis_index("c")
    sid = lax.axis_index("s")
    wid = sid * _NC + cid

    cp_s = pltpu.async_copy(s_hbm.at[pl.ds(wid * _ROWS_W, _ROWS_W)], s_v,
                            sem_in)
    cp_i = pltpu.async_copy(idx_hbm.at[pl.ds(wid * _ROWS_W, _ROWS_W)], idx_v,
                            sem_in)

    def _fill_ones(k, _):
        ones_v[pl.ds(k * _L, _L)] = jnp.full((_L,), 1.0, jnp.float32)
        return 0
    lax.fori_loop(0, _LANES // _L, _fill_ones, 0)

    @pl.when(sid == 0)
    def _init():
        def _fill_zero(k, _):
            zero_v[pl.ds(k * _L, _L)] = jnp.zeros((_L,), jnp.float32)
            return 0
        lax.fori_loop(0, _B // _L, _fill_zero, 0)
        pltpu.sync_copy(zero_v, acc_sh)
        pltpu.sync_copy(zero_v, cnt_sh)

    cp_s.wait()
    cp_i.wait()
    plsc.subcore_barrier()

    def _chunk(c, _):
        base = c * _FIRE
        for k in range(_FIRE):
            pltpu.async_copy(s_v.at[base + k],
                             acc_sh.at[idx_v.at[base + k]], sem_sc, add=True)
            pltpu.async_copy(ones_v,
                             cnt_sh.at[idx_v.at[base + k]], sem_sc, add=True)
        pltpu.make_async_copy(s_hbm.at[pl.ds(0, 2 * _FIRE)],
                              s_v.at[pl.ds(0, 2 * _FIRE)], sem_sc).wait()
        return 0
    lax.fori_loop(0, _ROWS_W // _FIRE, _chunk, 0)

    plsc.subcore_barrier()

    @pl.when(sid == 0)
    def _publish():
        pltpu.sync_copy(acc_sh, acc_out.at[cid])
        pltpu.sync_copy(cnt_sh, cnt_out.at[cid])


def _segment_partials(s2d, idx2d):
    mesh = plsc.VectorSubcoreMesh(
        core_axis_name="c", subcore_axis_name="s",
        num_cores=_NC, num_subcores=_NS)
    fn = functools.partial(
        pl.kernel,
        out_type=[jax.ShapeDtypeStruct((_NC, _B), jnp.float32),
                  jax.ShapeDtypeStruct((_NC, _B), jnp.float32)],
        mesh=mesh,
        scratch_types=[
            pltpu.VMEM((_ROWS_W, _LANES), jnp.float32),
            pltpu.VMEM((_ROWS_W, _LANES), jnp.int32),
            pltpu.VMEM((_LANES,), jnp.float32),
            pltpu.VMEM((_B,), jnp.float32),
            pltpu.VMEM_SHARED((_B,), jnp.float32),
            pltpu.VMEM_SHARED((_B,), jnp.float32),
            pltpu.SemaphoreType.DMA,
            pltpu.SemaphoreType.DMA,
        ],
    )(_seg_body)
    return fn(s2d, idx2d)


def _combine_body(acc, cnt, psv, smv, out):
    seg = acc[0:1, :] + acc[1:2, :]
    c = jnp.maximum(cnt[0:1, :] + cnt[1:2, :], 1.0)
    part = jnp.sum(seg / c)
    d = psv[...] - smv[...]
    errl = jnp.sum(d * d) / (smv.shape[0] * smv.shape[1])
    out[0, 0] = part / _B + errl


def _combine(acc, cnt, psv, smv):
    return pl.pallas_call(
        _combine_body,
        in_specs=[pl.BlockSpec(memory_space=pltpu.VMEM)] * 4,
        out_specs=pl.BlockSpec(memory_space=pltpu.SMEM),
        out_shape=jax.ShapeDtypeStruct((1, 1), jnp.float32),
    )(acc, cnt, psv, smv)


def kernel(pred_eps_x, target_eps_x, weights_x, pred_eps_h, eps_h,
           pred_symmetric_vector, symmetric_matrix_vector, batch_idx):
    s2d = _per_atom_sums(pred_eps_h, eps_h, pred_eps_x, target_eps_x,
                         weights_x)
    idx2d = batch_idx.reshape(_ROWS_TOTAL, _LANES)
    acc, cnt = _segment_partials(s2d, idx2d)
    out = _combine(acc, cnt, pred_symmetric_vector, symmetric_matrix_vector)
    return out[0, 0]


# speedup vs baseline: 13.3787x; 1.0866x over previous
import functools

import jax
import jax.numpy as jnp
from jax import lax
from jax.experimental import pallas as pl
from jax.experimental.pallas import tpu as pltpu
from jax.experimental.pallas import tpu_sc as plsc

_B = 2048
_N = 262144
_H = 100

_ROWS = 8192


def _rowsum_body(phT, ehT, pxT, txT, wT, out):
    dh = ehT[...] - phT[...]
    sh = jnp.sum(dh * dh, axis=0)
    dx = txT[...] - pxT[...]
    sx = jnp.sum(dx * dx, axis=0) * wT[0, :]
    out[...] = (sh + sx).reshape(_ROWS // 128, 128)


def _per_atom_sums(ph, eh, px, tx, w):
    return pl.pallas_call(
        _rowsum_body,
        grid=(_N // _ROWS,),
        in_specs=[
            pl.BlockSpec((_H, _ROWS), lambda i: (0, i)),
            pl.BlockSpec((_H, _ROWS), lambda i: (0, i)),
            pl.BlockSpec((3, _ROWS), lambda i: (0, i)),
            pl.BlockSpec((3, _ROWS), lambda i: (0, i)),
            pl.BlockSpec((1, _ROWS), lambda i: (0, i)),
        ],
        out_specs=pl.BlockSpec((_ROWS // 128, 128), lambda i: (i, 0)),
        out_shape=jax.ShapeDtypeStruct((_N // 128, 128), jnp.float32),
        compiler_params=pltpu.CompilerParams(
            dimension_semantics=("arbitrary",)),
    )(ph.T, eh.T, px.T, tx.T, w.T)


_NC, _NS, _L = 2, 16, 16
_NW = _NC * _NS
_LANES = 128
_ROWS_TOTAL = _N // _LANES
_ROWS_W = _ROWS_TOTAL // _NW


_FIRE = 8


_CHUNK = _N // _NW


def _seg_body(s_hbm, idx_hbm, acc_out, cnt_out,
              s_v, idx_v, ones_v, zero_v, acc_sh, cnt_sh, sem_in, sem_sc):
    cid = lax.axis_index("c")
    sid = lax.axis_index("s")
    wid = sid * _NC + cid

    cp_s = pltpu.async_copy(s_hbm.at[pl.ds(wid * _CHUNK, _CHUNK)], s_v,
                            sem_in)
    cp_i = pltpu.async_copy(idx_hbm.at[pl.ds(wid * _CHUNK, _CHUNK)], idx_v,
                            sem_in)

    def _fill_ones(k, _):
        ones_v[pl.ds(k * _L, _L)] = jnp.full((_L,), 1.0, jnp.float32)
        return 0
    lax.fori_loop(0, _CHUNK // _L, _fill_ones, 0)

    @pl.when(sid == 0)
    def _init():
        def _fill_zero(k, _):
            zero_v[pl.ds(k * _L, _L)] = jnp.zeros((_L,), jnp.float32)
            return 0
        lax.fori_loop(0, _B // _L, _fill_zero, 0)
        pltpu.sync_copy(zero_v, acc_sh)
        pltpu.sync_copy(zero_v, cnt_sh)

    cp_s.wait()
    cp_i.wait()
    plsc.subcore_barrier()

    cp_a = pltpu.async_copy(s_v, acc_sh.at[idx_v], sem_sc, add=True)
    cp_c = pltpu.async_copy(ones_v, cnt_sh.at[idx_v], sem_sc, add=True)
    cp_a.wait()
    cp_c.wait()

    plsc.subcore_barrier()

    @pl.when(sid == 0)
    def _publish():
        pltpu.sync_copy(acc_sh, acc_out.at[cid])
        pltpu.sync_copy(cnt_sh, cnt_out.at[cid])


def _segment_partials(s2d, idx2d):
    mesh = plsc.VectorSubcoreMesh(
        core_axis_name="c", subcore_axis_name="s",
        num_cores=_NC, num_subcores=_NS)
    fn = functools.partial(
        pl.kernel,
        out_type=[jax.ShapeDtypeStruct((_NC, _B), jnp.float32),
                  jax.ShapeDtypeStruct((_NC, _B), jnp.float32)],
        mesh=mesh,
        scratch_types=[
            pltpu.VMEM((_CHUNK,), jnp.float32),
            pltpu.VMEM((_CHUNK,), jnp.int32),
            pltpu.VMEM((_CHUNK,), jnp.float32),
            pltpu.VMEM((_B,), jnp.float32),
            pltpu.VMEM_SHARED((_B,), jnp.float32),
            pltpu.VMEM_SHARED((_B,), jnp.float32),
            pltpu.SemaphoreType.DMA,
            pltpu.SemaphoreType.DMA,
        ],
    )(_seg_body)
    return fn(s2d, idx2d)


def _combine_body(acc, cnt, psv, smv, out):
    seg = acc[0:1, :] + acc[1:2, :]
    c = jnp.maximum(cnt[0:1, :] + cnt[1:2, :], 1.0)
    part = jnp.sum(seg / c)
    d = psv[...] - smv[...]
    errl = jnp.sum(d * d) / (smv.shape[0] * smv.shape[1])
    out[0, 0] = part / _B + errl


def _combine(acc, cnt, psv, smv):
    return pl.pallas_call(
        _combine_body,
        in_specs=[pl.BlockSpec(memory_space=pltpu.VMEM)] * 4,
        out_specs=pl.BlockSpec(memory_space=pltpu.SMEM),
        out_shape=jax.ShapeDtypeStruct((1, 1), jnp.float32),
    )(acc, cnt, psv, smv)


def kernel(pred_eps_x, target_eps_x, weights_x, pred_eps_h, eps_h,
           pred_symmetric_vector, symmetric_matrix_vector, batch_idx):
    s2d = _per_atom_sums(pred_eps_h, eps_h, pred_eps_x, target_eps_x,
                         weights_x)
    acc, cnt = _segment_partials(s2d.reshape(_N), batch_idx)
    out = _combine(acc, cnt, pred_symmetric_vector, symmetric_matrix_vector)
    return out[0, 0]


# speedup vs baseline: 17.4080x; 1.3012x over previous
import functools

import jax
import jax.numpy as jnp
from jax import lax
from jax.experimental import pallas as pl
from jax.experimental.pallas import tpu as pltpu
from jax.experimental.pallas import tpu_sc as plsc

_B = 2048
_N = 262144
_H = 100

_ROWS = 8192


def _rowsum_body(phT, ehT, pxT, txT, wT, out):
    dh = ehT[...] - phT[...]
    sh = jnp.sum(dh * dh, axis=0)
    dx = txT[...] - pxT[...]
    sx = jnp.sum(dx * dx, axis=0) * wT[0, :]
    srows = (sh + sx).reshape(_ROWS // 128, 128)
    ri = lax.broadcasted_iota(jnp.int32, (128, 128), 0)
    ci = lax.broadcasted_iota(jnp.int32, (128, 128), 1)
    tri = (ri <= ci).astype(jnp.float32)
    out[...] = lax.dot_general(
        srows, tri, (((1,), (0,)), ((), ())),
        preferred_element_type=jnp.float32,
        precision=lax.Precision.HIGHEST)


def _per_atom_sums(ph, eh, px, tx, w):
    return pl.pallas_call(
        _rowsum_body,
        grid=(_N // _ROWS,),
        in_specs=[
            pl.BlockSpec((_H, _ROWS), lambda i: (0, i)),
            pl.BlockSpec((_H, _ROWS), lambda i: (0, i)),
            pl.BlockSpec((3, _ROWS), lambda i: (0, i)),
            pl.BlockSpec((3, _ROWS), lambda i: (0, i)),
            pl.BlockSpec((1, _ROWS), lambda i: (0, i)),
        ],
        out_specs=pl.BlockSpec((_ROWS // 128, 128), lambda i: (i, 0)),
        out_shape=jax.ShapeDtypeStruct((_N // 128, 128), jnp.float32),
        compiler_params=pltpu.CompilerParams(
            dimension_semantics=("arbitrary",)),
    )(ph.T, eh.T, px.T, tx.T, w.T)


_NC, _NS, _L = 2, 16, 16
_NW = _NC * _NS
_LANES = 128
_ROWS_TOTAL = _N // _LANES
_ROWS_W = _ROWS_TOTAL // _NW


_FIRE = 8


_CHUNK = _N // _NW
_NROWS = _CHUNK // 128
_MAXB = _B + 32


def _seg_body(rc_hbm, idx_hbm, acc_out, cnt_out,
              rc_v, idx_v, rowbase_v, bpos_v, bval_v, ssum_v, scnt_v,
              zero_v, acc_sh, cnt_sh, sem_in, sem_sc):
    cid = lax.axis_index("c")
    sid = lax.axis_index("s")
    wid = sid * _NC + cid
    iota = lax.iota(jnp.int32, _L)

    cp_s = pltpu.async_copy(rc_hbm.at[pl.ds(wid * _CHUNK, _CHUNK)], rc_v,
                            sem_in)
    cp_i = pltpu.async_copy(idx_hbm.at[pl.ds(wid * _CHUNK, _CHUNK)], idx_v,
                            sem_in)

    @pl.when(sid == 0)
    def _init():
        def _fill_zero(k, _):
            zero_v[pl.ds(k * _L, _L)] = jnp.zeros((_L,), jnp.float32)
            return 0
        lax.fori_loop(0, _B // _L, _fill_zero, 0)
        pltpu.sync_copy(zero_v, acc_sh)
        pltpu.sync_copy(zero_v, cnt_sh)

    cp_s.wait()
    cp_i.wait()

    def _rowbase(m, carry):
        rt = plsc.load_gather(rc_v, [(m * _L + iota) * 128 + 127])
        rowbase_v[pl.ds(m * _L, _L)] = plsc.cumsum(rt) - rt + carry
        return carry + jnp.sum(rt)
    lax.fori_loop(0, _NROWS // _L, _rowbase, jnp.float32(0.0))

    def _bounds(j, off):
        v = idx_v[pl.ds(j * _L, _L)]
        prev = plsc.load_gather(idx_v, [jnp.maximum(j * _L - 1 + iota, 0)])
        mask = (v != prev) | ((iota == 0) & (j == 0))
        plsc.store_compressed(bpos_v.at[pl.ds(off, _L)], j * _L + iota,
                              mask=mask)
        plsc.store_compressed(bval_v.at[pl.ds(off, _L)], v, mask=mask)
        return off + jnp.max(plsc.all_reduce_population_count(mask))
    nb = lax.fori_loop(0, _CHUNK // _L, _bounds, jnp.int32(0))
    bpos_v[pl.ds(nb, _L)] = jnp.full((_L,), _CHUNK, jnp.int32)

    def _runs(t, _):
        kk = t * _L
        pv = plsc.load_gather(bpos_v, [kk + iota])
        pv1 = plsc.load_gather(bpos_v, [kk + 1 + iota])
        val = plsc.load_gather(bval_v, [kk + iota])
        valid = (kk + iota) < nb
        q1 = jnp.clip(pv1 - 1, 0, _CHUNK - 1)
        q0 = jnp.clip(pv - 1, 0, _CHUNK - 1)
        cs1 = plsc.load_gather(rc_v, [q1]) \
            + plsc.load_gather(rowbase_v, [q1 >> 7])
        cs0 = plsc.load_gather(rc_v, [q0]) \
            + plsc.load_gather(rowbase_v, [q0 >> 7])
        cs0 = jnp.where(pv > 0, cs0, 0.0)
        ssum_v[pl.ds(kk, _L)] = jnp.where(valid, cs1 - cs0, 0.0)
        scnt_v[pl.ds(kk, _L)] = jnp.where(
            valid, (pv1 - pv).astype(jnp.float32), 0.0)
        bval_v[pl.ds(kk, _L)] = jnp.where(valid, val, 0)
        return 0
    n16 = (nb + _L - 1) >> 4
    lax.fori_loop(0, n16, _runs, 0)

    def _pad(m, _):
        kk = m * _L
        ssum_v[pl.ds(kk, _L)] = jnp.zeros((_L,), jnp.float32)
        scnt_v[pl.ds(kk, _L)] = jnp.zeros((_L,), jnp.float32)
        bval_v[pl.ds(kk, _L)] = jnp.zeros((_L,), jnp.int32)
        return 0
    lax.fori_loop(n16, ((nb + 127) >> 7) << 3, _pad, 0)

    plsc.subcore_barrier()

    def _scatter(t, _):
        pltpu.sync_copy(ssum_v.at[pl.ds(t * 128, 128)],
                        acc_sh.at[bval_v.at[pl.ds(t * 128, 128)]], add=True)
        pltpu.sync_copy(scnt_v.at[pl.ds(t * 128, 128)],
                        cnt_sh.at[bval_v.at[pl.ds(t * 128, 128)]], add=True)
        return 0
    lax.fori_loop(0, (nb + 127) >> 7, _scatter, 0)

    plsc.subcore_barrier()

    @pl.when(sid == 0)
    def _publish():
        pltpu.sync_copy(acc_sh, acc_out.at[cid])
        pltpu.sync_copy(cnt_sh, cnt_out.at[cid])


def _segment_partials(s2d, idx2d):
    mesh = plsc.VectorSubcoreMesh(
        core_axis_name="c", subcore_axis_name="s",
        num_cores=_NC, num_subcores=_NS)
    fn = functools.partial(
        pl.kernel,
        out_type=[jax.ShapeDtypeStruct((_NC, _B), jnp.float32),
                  jax.ShapeDtypeStruct((_NC, _B), jnp.float32)],
        mesh=mesh,
        scratch_types=[
            pltpu.VMEM((_CHUNK,), jnp.float32),
            pltpu.VMEM((_CHUNK,), jnp.int32),
            pltpu.VMEM((_NROWS + _L,), jnp.float32),
            pltpu.VMEM((_MAXB,), jnp.int32),
            pltpu.VMEM((_MAXB,), jnp.int32),
            pltpu.VMEM((_MAXB,), jnp.float32),
            pltpu.VMEM((_MAXB,), jnp.float32),
            pltpu.VMEM((_B,), jnp.float32),
            pltpu.VMEM_SHARED((_B,), jnp.float32),
            pltpu.VMEM_SHARED((_B,), jnp.float32),
            pltpu.SemaphoreType.DMA,
            pltpu.SemaphoreType.DMA,
        ],
        compiler_params=pltpu.CompilerParams(needs_layout_passes=False),
    )(_seg_body)
    return fn(s2d, idx2d)


def _combine_body(acc, cnt, psv, smv, out):
    seg = acc[0:1, :] + acc[1:2, :]
    c = jnp.maximum(cnt[0:1, :] + cnt[1:2, :], 1.0)
    part = jnp.sum(seg / c)
    d = psv[...] - smv[...]
    errl = jnp.sum(d * d) / (smv.shape[0] * smv.shape[1])
    out[0, 0] = part / _B + errl


def _combine(acc, cnt, psv, smv):
    return pl.pallas_call(
        _combine_body,
        in_specs=[pl.BlockSpec(memory_space=pltpu.VMEM)] * 4,
        out_specs=pl.BlockSpec(memory_space=pltpu.SMEM),
        out_shape=jax.ShapeDtypeStruct((1, 1), jnp.float32),
    )(acc, cnt, psv, smv)


def kernel(pred_eps_x, target_eps_x, weights_x, pred_eps_h, eps_h,
           pred_symmetric_vector, symmetric_matrix_vector, batch_idx):
    s2d = _per_atom_sums(pred_eps_h, eps_h, pred_eps_x, target_eps_x,
                         weights_x)
    acc, cnt = _segment_partials(s2d.reshape(_N), batch_idx)
    out = _combine(acc, cnt, pred_symmetric_vector, symmetric_matrix_vector)
    return out[0, 0]


# speedup vs baseline: 17.4100x; 1.0001x over previous
import functools

import jax
import jax.numpy as jnp
from jax import lax
from jax.experimental import pallas as pl
from jax.experimental.pallas import tpu as pltpu
from jax.experimental.pallas import tpu_sc as plsc

_B = 2048
_N = 262144
_H = 100

_ROWS = 8192


def _rowsum_body(phT, ehT, pxT, txT, wT, out):
    dh = ehT[...] - phT[...]
    sh = jnp.sum(dh * dh, axis=0)
    dx = txT[...] - pxT[...]
    sx = jnp.sum(dx * dx, axis=0) * wT[0, :]
    srows = (sh + sx).reshape(_ROWS // 128, 128)
    ri = lax.broadcasted_iota(jnp.int32, (128, 128), 0)
    ci = lax.broadcasted_iota(jnp.int32, (128, 128), 1)
    tri = (ri <= ci).astype(jnp.float32)
    out[...] = lax.dot_general(
        srows, tri, (((1,), (0,)), ((), ())),
        preferred_element_type=jnp.float32,
        precision=lax.Precision.HIGHEST)


def _per_atom_sums(ph, eh, px, tx, w):
    return pl.pallas_call(
        _rowsum_body,
        grid=(_N // _ROWS,),
        in_specs=[
            pl.BlockSpec((_H, _ROWS), lambda i: (0, i)),
            pl.BlockSpec((_H, _ROWS), lambda i: (0, i)),
            pl.BlockSpec((3, _ROWS), lambda i: (0, i)),
            pl.BlockSpec((3, _ROWS), lambda i: (0, i)),
            pl.BlockSpec((1, _ROWS), lambda i: (0, i)),
        ],
        out_specs=pl.BlockSpec((_ROWS // 128, 128), lambda i: (i, 0)),
        out_shape=jax.ShapeDtypeStruct((_N // 128, 128), jnp.float32),
        compiler_params=pltpu.CompilerParams(
            dimension_semantics=("arbitrary",)),
    )(ph.T, eh.T, px.T, tx.T, w.T)


_NC, _NS, _L = 2, 16, 16
_NW = _NC * _NS
_CHUNK = _N // _NW
_NROWS = _CHUNK // 128
_MAXB = _B + 32


def _seg_body(rc_hbm, idx_hbm, acc_out, cnt_out,
              rc_v, idx_v, rowbase_v, bpos_v, bval_v, ssum_v, scnt_v,
              zero_v, acc_sh, cnt_sh, sem_in, sem_sc):
    cid = lax.axis_index("c")
    sid = lax.axis_index("s")
    wid = sid * _NC + cid
    iota = lax.iota(jnp.int32, _L)

    cp_s = pltpu.async_copy(rc_hbm.at[pl.ds(wid * _CHUNK, _CHUNK)], rc_v,
                            sem_in)
    cp_i = pltpu.async_copy(idx_hbm.at[pl.ds(wid * _CHUNK, _CHUNK)], idx_v,
                            sem_in)

    @pl.when(sid == 0)
    def _init():
        def _fill_zero(k, _):
            zero_v[pl.ds(k * _L, _L)] = jnp.zeros((_L,), jnp.float32)
            return 0
        lax.fori_loop(0, _B // _L, _fill_zero, 0)
        pltpu.sync_copy(zero_v, acc_sh)
        pltpu.sync_copy(zero_v, cnt_sh)

    cp_s.wait()
    cp_i.wait()

    def _rowbase(m, carry):
        rt = plsc.load_gather(rc_v, [(m * _L + iota) * 128 + 127])
        rowbase_v[pl.ds(m * _L, _L)] = plsc.cumsum(rt) - rt + carry
        return carry + jnp.sum(rt)
    lax.fori_loop(0, _NROWS // _L, _rowbase, jnp.float32(0.0))

    def _bounds(j, off):
        v = idx_v[pl.ds(j * _L, _L)]
        prev = plsc.load_gather(idx_v, [jnp.maximum(j * _L - 1 + iota, 0)])
        mask = (v != prev) | ((iota == 0) & (j == 0))
        plsc.store_compressed(bpos_v.at[pl.ds(off, _L)], j * _L + iota,
                              mask=mask)
        plsc.store_compressed(bval_v.at[pl.ds(off, _L)], v, mask=mask)
        return off + jnp.max(plsc.all_reduce_population_count(mask))
    nb = lax.fori_loop(0, _CHUNK // _L, _bounds, jnp.int32(0))
    bpos_v[pl.ds(nb, _L)] = jnp.full((_L,), _CHUNK, jnp.int32)

    def _runs(t, _):
        kk = t * _L
        pv = plsc.load_gather(bpos_v, [kk + iota])
        pv1 = plsc.load_gather(bpos_v, [kk + 1 + iota])
        val = plsc.load_gather(bval_v, [kk + iota])
        valid = (kk + iota) < nb
        q1 = jnp.clip(pv1 - 1, 0, _CHUNK - 1)
        q0 = jnp.clip(pv - 1, 0, _CHUNK - 1)
        cs1 = plsc.load_gather(rc_v, [q1]) \
            + plsc.load_gather(rowbase_v, [q1 >> 7])
        cs0 = plsc.load_gather(rc_v, [q0]) \
            + plsc.load_gather(rowbase_v, [q0 >> 7])
        cs0 = jnp.where(pv > 0, cs0, 0.0)
        ssum_v[pl.ds(kk, _L)] = jnp.where(valid, cs1 - cs0, 0.0)
        scnt_v[pl.ds(kk, _L)] = jnp.where(
            valid, (pv1 - pv).astype(jnp.float32), 0.0)
        bval_v[pl.ds(kk, _L)] = jnp.where(valid, val, 0)
        return 0
    n16 = (nb + _L - 1) >> 4
    lax.fori_loop(0, n16, _runs, 0)

    def _pad(m, _):
        kk = m * _L
        ssum_v[pl.ds(kk, _L)] = jnp.zeros((_L,), jnp.float32)
        scnt_v[pl.ds(kk, _L)] = jnp.zeros((_L,), jnp.float32)
        bval_v[pl.ds(kk, _L)] = jnp.zeros((_L,), jnp.int32)
        return 0
    lax.fori_loop(n16, ((nb + 127) >> 7) << 3, _pad, 0)

    plsc.subcore_barrier()

    def _scatter(t, _):
        pltpu.sync_copy(ssum_v.at[pl.ds(t * 128, 128)],
                        acc_sh.at[bval_v.at[pl.ds(t * 128, 128)]], add=True)
        pltpu.sync_copy(scnt_v.at[pl.ds(t * 128, 128)],
                        cnt_sh.at[bval_v.at[pl.ds(t * 128, 128)]], add=True)
        return 0
    lax.fori_loop(0, (nb + 127) >> 7, _scatter, 0)

    plsc.subcore_barrier()

    @pl.when(sid == 0)
    def _publish():
        pltpu.sync_copy(acc_sh, acc_out.at[cid])
        pltpu.sync_copy(cnt_sh, cnt_out.at[cid])


def _segment_partials(s2d, idx2d):
    mesh = plsc.VectorSubcoreMesh(
        core_axis_name="c", subcore_axis_name="s",
        num_cores=_NC, num_subcores=_NS)
    fn = functools.partial(
        pl.kernel,
        out_type=[jax.ShapeDtypeStruct((_NC, _B), jnp.float32),
                  jax.ShapeDtypeStruct((_NC, _B), jnp.float32)],
        mesh=mesh,
        scratch_types=[
            pltpu.VMEM((_CHUNK,), jnp.float32),
            pltpu.VMEM((_CHUNK,), jnp.int32),
            pltpu.VMEM((_NROWS + _L,), jnp.float32),
            pltpu.VMEM((_MAXB,), jnp.int32),
            pltpu.VMEM((_MAXB,), jnp.int32),
            pltpu.VMEM((_MAXB,), jnp.float32),
            pltpu.VMEM((_MAXB,), jnp.float32),
            pltpu.VMEM((_B,), jnp.float32),
            pltpu.VMEM_SHARED((_B,), jnp.float32),
            pltpu.VMEM_SHARED((_B,), jnp.float32),
            pltpu.SemaphoreType.DMA,
            pltpu.SemaphoreType.DMA,
        ],
        compiler_params=pltpu.CompilerParams(needs_layout_passes=False),
    )(_seg_body)
    return fn(s2d, idx2d)


def _combine_body(acc, cnt, psv, smv, out):
    seg = acc[0:1, :] + acc[1:2, :]
    c = jnp.maximum(cnt[0:1, :] + cnt[1:2, :], 1.0)
    part = jnp.sum(seg / c)
    d = psv[...] - smv[...]
    errl = jnp.sum(d * d) / (smv.shape[0] * smv.shape[1])
    out[0, 0] = part / _B + errl


def _combine(acc, cnt, psv, smv):
    return pl.pallas_call(
        _combine_body,
        in_specs=[pl.BlockSpec(memory_space=pltpu.VMEM)] * 4,
        out_specs=pl.BlockSpec(memory_space=pltpu.SMEM),
        out_shape=jax.ShapeDtypeStruct((1, 1), jnp.float32),
    )(acc, cnt, psv, smv)


def kernel(pred_eps_x, target_eps_x, weights_x, pred_eps_h, eps_h,
           pred_symmetric_vector, symmetric_matrix_vector, batch_idx):
    s2d = _per_atom_sums(pred_eps_h, eps_h, pred_eps_x, target_eps_x,
                         weights_x)
    acc, cnt = _segment_partials(s2d.reshape(_N), batch_idx)
    out = _combine(acc, cnt, pred_symmetric_vector, symmetric_matrix_vector)
    return out[0, 0]
